# baseline (speedup 1.0000x reference)
import functools

import jax
import jax.numpy as jnp
from jax import lax
from jax.experimental import pallas as pl
from jax.experimental.pallas import tpu as pltpu
from jax.experimental.pallas import tpu_sc as plsc

NC = 2
NS = 16
NW = NC * NS
LANES = 16
CH = 48
NBUF = 4
BN = 1024


def _sc_mesh():
    return plsc.VectorSubcoreMesh(
        core_axis_name="c", subcore_axis_name="s",
        num_cores=NC, num_subcores=NS)


def _make_deg_kernel(n_pad, nchunk):
    rpt = n_pad // NS

    @functools.partial(
        pl.kernel,
        out_type=jax.ShapeDtypeStruct((NC, n_pad), jnp.float32),
        mesh=_sc_mesh(),
        scratch_types=[
            pltpu.VMEM((nchunk, CH), jnp.int32),
            pltpu.VMEM((CH,), jnp.float32),
            pltpu.VMEM((rpt,), jnp.float32),
            pltpu.VMEM_SHARED((n_pad,), jnp.float32),
            pltpu.SemaphoreType.DMA,
            pltpu.SemaphoreType.DMA,
        ],
    )
    def deg_kernel(dst_hbm, ones_hbm, degp_hbm, dstv, ones_v, zv, deg_sh,
                   dsem_a, dsem_b):
        c = lax.axis_index("c")
        s = lax.axis_index("s")
        wid = s * NC + c
        pltpu.sync_copy(dst_hbm.at[wid], dstv)
        pltpu.sync_copy(ones_hbm, ones_v)
        for i in range(rpt // LANES):
            zv[pl.ds(i * LANES, LANES)] = jnp.zeros((LANES,), jnp.float32)
        row0 = pl.multiple_of(s * rpt, 8)
        pltpu.sync_copy(zv, deg_sh.at[pl.ds(row0, rpt)])
        plsc.subcore_barrier()

        def d_desc(ci, sem):
            return pltpu.make_async_copy(ones_v, deg_sh.at[dstv.at[ci]], sem)

        def body(k, carry):
            c0 = k * 2
            d_desc(c0, dsem_a).start(add=True)
            d_desc(c0 + 1, dsem_b).start(add=True)
            d_desc(c0, dsem_a).wait()
            d_desc(c0 + 1, dsem_b).wait()
            return carry

        lax.fori_loop(0, nchunk // 2, body, 0)
        for ci in range(nchunk - nchunk % 2, nchunk):
            pltpu.sync_copy(ones_v, deg_sh.at[dstv.at[ci]], add=True)
        plsc.subcore_barrier()
        pltpu.sync_copy(deg_sh.at[pl.ds(row0, rpt)],
                        degp_hbm.at[c, pl.ds(row0, rpt)])

    return deg_kernel


CPB = 12


def _make_agg_kernel(n_pad, nout, epw, nchunk):
    rpt = n_pad // NS
    nblk = nchunk // CPB
    be = CPB * CH
    assert nblk >= 4 and nblk % 2 == 0 and CPB % NBUF == 0
    depth = NBUF - 1

    @functools.partial(
        pl.kernel,
        out_type=jax.ShapeDtypeStruct((NC, n_pad, nout), jnp.float32),
        mesh=_sc_mesh(),
        scratch_types=(
            [pltpu.VMEM((be,), jnp.int32)] * 2 +
            [pltpu.VMEM((CPB, CH), jnp.int32)] * 2 +
            [pltpu.VMEM((CH, nout), jnp.float32)] * NBUF +
            [pltpu.VMEM_SHARED((n_pad, nout), jnp.float32)] +
            [pltpu.SemaphoreType.DMA] * (2 * NBUF + 2)
        ),
    )
    def agg_kernel(g_hbm, src_hbm, dst_hbm, zrows_hbm, qp_hbm, *scr):
        srcv = list(scr[0:2])
        dstv = list(scr[2:4])
        rows = list(scr[4:4 + NBUF])
        acc = scr[4 + NBUF]
        gsem = list(scr[5 + NBUF:5 + 2 * NBUF])
        ssem = list(scr[5 + 2 * NBUF:5 + 3 * NBUF])
        isem = list(scr[5 + 3 * NBUF:7 + 3 * NBUF])
        c = lax.axis_index("c")
        s = lax.axis_index("s")
        wid = s * NC + c
        row0 = pl.multiple_of(s * rpt, 8)

        pltpu.sync_copy(zrows_hbm, acc.at[pl.ds(row0, rpt)])
        plsc.subcore_barrier()

        def g_desc(i, p, x):
            return pltpu.make_async_copy(
                g_hbm.at[srcv[p].at[pl.ds(i * CH, CH)]], rows[x], gsem[x])

        def s_desc(i, p, x):
            return pltpu.make_async_copy(
                rows[x], acc.at[dstv[p].at[i]], ssem[x])

        def stage_descs(sb, p):
            sbase = pl.multiple_of(wid * epw + sb * be, 8)
            return (pltpu.make_async_copy(
                        src_hbm.at[pl.ds(sbase, be)], srcv[p], isem[p]),
                    pltpu.make_async_copy(
                        dst_hbm.at[wid, sb], dstv[p], isem[p]))

        def block_body(sb, p, first=False, last=False):
            if not last:
                for d in stage_descs(sb + 1, 1 - p):
                    d.start()
            for i in range(CPB):
                x = i % NBUF
                g_desc(i, p, x).wait()
                s_desc(i, p, x).start(add=True)
                if not (first and i == 0):
                    s_desc(0, p, (i + NBUF - 1) % NBUF).wait()
                if i <= CPB - 1 - depth:
                    g_desc(i + depth, p, (i + depth) % NBUF).start()
                elif not last:
                    if i == CPB - depth:
                        for d in stage_descs(sb + 1, 1 - p):
                            d.wait()
                    g_desc(i + depth - CPB, 1 - p, (i + depth) % NBUF).start()

        for d in stage_descs(0, 0):
            d.start()
        for d in stage_descs(0, 0):
            d.wait()
        for k in range(depth):
            g_desc(k, 0, k % NBUF).start()

        block_body(0, 0, first=True)

        def pair_body(m, carry):
            block_body(2 * m + 1, 1)
            block_body(2 * m + 2, 0)
            return carry

        lax.fori_loop(0, (nblk - 2) // 2, pair_body, 0)
        block_body(nblk - 1, (nblk - 1) % 2, last=True)
        s_desc(0, 0, (CPB - 1) % NBUF).wait()

        plsc.subcore_barrier()
        pltpu.sync_copy(acc.at[pl.ds(row0, rpt)],
                        qp_hbm.at[c, pl.ds(row0, rpt)])

    return agg_kernel


def _gscale_body(degp_ref, x_ref, w_ref, g_ref):
    deg = degp_ref[0, :] + degp_ref[1, :] + 1.0
    dinv = lax.rsqrt(deg)
    h = jnp.dot(x_ref[...], w_ref[...], preferred_element_type=jnp.float32)
    g_ref[...] = h * dinv[:, None]


def _combine_body(degp_ref, q_ref, g_ref, b_ref, o_ref):
    deg = degp_ref[0, :] + degp_ref[1, :] + 1.0
    dinv = lax.rsqrt(deg)
    ssum = q_ref[0] + q_ref[1] + g_ref[...]
    o_ref[...] = ssum * dinv[:, None] + b_ref[...]


def kernel(x, edge_index, W, b):
    n, nin = x.shape
    nout = W.shape[1]
    e = edge_index.shape[1]

    npm = max(BN, NS * 8)
    n_pad = -(-n // npm) * npm
    epw = -(-e // (NW * CPB * CH)) * (CPB * CH)
    ep = epw * NW
    nchunk = epw // CH
    nblk = nchunk // CPB

    src = edge_index[0].astype(jnp.int32)
    dst = edge_index[1].astype(jnp.int32)
    pad = ep - e
    if pad:
        extra = n + (jnp.arange(pad, dtype=jnp.int32) % (n_pad - n))
        src = jnp.concatenate([src, extra])
        dst = jnp.concatenate([dst, extra])
    dst_r = dst.reshape(NW, nchunk, CH)
    dst_r4 = dst.reshape(NW, nblk, CPB, CH)
    xp = jnp.pad(x, ((0, n_pad - n), (0, 0)))
    zrows = jnp.zeros((n_pad // NS, nout), jnp.float32)
    ones_ch = jnp.ones((CH,), jnp.float32)

    degp = _make_deg_kernel(n_pad, nchunk)(dst_r, ones_ch)

    g = pl.pallas_call(
        _gscale_body,
        grid=(n_pad // BN,),
        in_specs=[
            pl.BlockSpec((NC, BN), lambda i: (0, i)),
            pl.BlockSpec((BN, nin), lambda i: (i, 0)),
            pl.BlockSpec((nin, nout), lambda i: (0, 0)),
        ],
        out_specs=pl.BlockSpec((BN, nout), lambda i: (i, 0)),
        out_shape=jax.ShapeDtypeStruct((n_pad, nout), jnp.float32),
    )(degp, xp, W)

    qp = _make_agg_kernel(n_pad, nout, epw, nchunk)(g, src, dst_r4, zrows)

    outp = pl.pallas_call(
        _combine_body,
        grid=(n_pad // BN,),
        in_specs=[
            pl.BlockSpec((NC, BN), lambda i: (0, i)),
            pl.BlockSpec((NC, BN, nout), lambda i: (0, i, 0)),
            pl.BlockSpec((BN, nout), lambda i: (i, 0)),
            pl.BlockSpec((1, nout), lambda i: (0, 0)),
        ],
        out_specs=pl.BlockSpec((BN, nout), lambda i: (i, 0)),
        out_shape=jax.ShapeDtypeStruct((n_pad, nout), jnp.float32),
    )(degp, qp, g, b.reshape(1, nout))

    return outp[:n]


# speedup vs baseline: 1.0613x; 1.0613x over previous
import functools

import jax
import jax.numpy as jnp
from jax import lax
from jax.experimental import pallas as pl
from jax.experimental.pallas import tpu as pltpu
from jax.experimental.pallas import tpu_sc as plsc

NC = 2
NS = 16
NW = NC * NS
LANES = 16
CH = 72
NBUF = 3
BN = 1024


def _sc_mesh():
    return plsc.VectorSubcoreMesh(
        core_axis_name="c", subcore_axis_name="s",
        num_cores=NC, num_subcores=NS)


def _make_deg_kernel(n_pad, nchunk):
    rpt = n_pad // NS

    @functools.partial(
        pl.kernel,
        out_type=jax.ShapeDtypeStruct((NC, n_pad), jnp.float32),
        mesh=_sc_mesh(),
        scratch_types=[
            pltpu.VMEM((nchunk, CH), jnp.int32),
            pltpu.VMEM((CH,), jnp.float32),
            pltpu.VMEM((rpt,), jnp.float32),
            pltpu.VMEM_SHARED((n_pad,), jnp.float32),
            pltpu.SemaphoreType.DMA,
            pltpu.SemaphoreType.DMA,
        ],
    )
    def deg_kernel(dst_hbm, ones_hbm, degp_hbm, dstv, ones_v, zv, deg_sh,
                   dsem_a, dsem_b):
        c = lax.axis_index("c")
        s = lax.axis_index("s")
        wid = s * NC + c
        pltpu.sync_copy(dst_hbm.at[wid], dstv)
        pltpu.sync_copy(ones_hbm, ones_v)
        for i in range(rpt // LANES):
            zv[pl.ds(i * LANES, LANES)] = jnp.zeros((LANES,), jnp.float32)
        row0 = pl.multiple_of(s * rpt, 8)
        pltpu.sync_copy(zv, deg_sh.at[pl.ds(row0, rpt)])
        plsc.subcore_barrier()

        def d_desc(ci, sem):
            return pltpu.make_async_copy(ones_v, deg_sh.at[dstv.at[ci]], sem)

        def body(k, carry):
            c0 = k * 2
            d_desc(c0, dsem_a).start(add=True)
            d_desc(c0 + 1, dsem_b).start(add=True)
            d_desc(c0, dsem_a).wait()
            d_desc(c0 + 1, dsem_b).wait()
            return carry

        lax.fori_loop(0, nchunk // 2, body, 0)
        for ci in range(nchunk - nchunk % 2, nchunk):
            pltpu.sync_copy(ones_v, deg_sh.at[dstv.at[ci]], add=True)
        plsc.subcore_barrier()
        pltpu.sync_copy(deg_sh.at[pl.ds(row0, rpt)],
                        degp_hbm.at[c, pl.ds(row0, rpt)])

    return deg_kernel


CPB = 9


def _make_agg_kernel(n_pad, nout, epw, nchunk):
    rpt = n_pad // NS
    nblk = nchunk // CPB
    be = CPB * CH
    assert nblk >= 4 and nblk % 2 == 0 and CPB % NBUF == 0
    depth = NBUF - 1

    @functools.partial(
        pl.kernel,
        out_type=jax.ShapeDtypeStruct((NC, n_pad, nout), jnp.float32),
        mesh=_sc_mesh(),
        scratch_types=(
            [pltpu.VMEM((be,), jnp.int32)] * 2 +
            [pltpu.VMEM((CPB, CH), jnp.int32)] * 2 +
            [pltpu.VMEM((CH, nout), jnp.float32)] * NBUF +
            [pltpu.VMEM_SHARED((n_pad, nout), jnp.float32)] +
            [pltpu.SemaphoreType.DMA] * (2 * NBUF + 2)
        ),
    )
    def agg_kernel(g_hbm, src_hbm, dst_hbm, zrows_hbm, qp_hbm, *scr):
        srcv = list(scr[0:2])
        dstv = list(scr[2:4])
        rows = list(scr[4:4 + NBUF])
        acc = scr[4 + NBUF]
        gsem = list(scr[5 + NBUF:5 + 2 * NBUF])
        ssem = list(scr[5 + 2 * NBUF:5 + 3 * NBUF])
        isem = list(scr[5 + 3 * NBUF:7 + 3 * NBUF])
        c = lax.axis_index("c")
        s = lax.axis_index("s")
        wid = s * NC + c
        row0 = pl.multiple_of(s * rpt, 8)

        pltpu.sync_copy(zrows_hbm, acc.at[pl.ds(row0, rpt)])
        plsc.subcore_barrier()

        def g_desc(i, p, x):
            return pltpu.make_async_copy(
                g_hbm.at[srcv[p].at[pl.ds(i * CH, CH)]], rows[x], gsem[x])

        def s_desc(i, p, x):
            return pltpu.make_async_copy(
                rows[x], acc.at[dstv[p].at[i]], ssem[x])

        def stage_descs(sb, p):
            sbase = pl.multiple_of(wid * epw + sb * be, 8)
            return (pltpu.make_async_copy(
                        src_hbm.at[pl.ds(sbase, be)], srcv[p], isem[p]),
                    pltpu.make_async_copy(
                        dst_hbm.at[wid, sb], dstv[p], isem[p]))

        def block_body(sb, p, first=False, last=False):
            if not last:
                for d in stage_descs(sb + 1, 1 - p):
                    d.start()
            for i in range(CPB):
                x = i % NBUF
                g_desc(i, p, x).wait()
                s_desc(i, p, x).start(add=True)
                if not (first and i == 0):
                    s_desc(0, p, (i + NBUF - 1) % NBUF).wait()
                if i <= CPB - 1 - depth:
                    g_desc(i + depth, p, (i + depth) % NBUF).start()
                elif not last:
                    if i == CPB - depth:
                        for d in stage_descs(sb + 1, 1 - p):
                            d.wait()
                    g_desc(i + depth - CPB, 1 - p, (i + depth) % NBUF).start()

        for d in stage_descs(0, 0):
            d.start()
        for d in stage_descs(0, 0):
            d.wait()
        for k in range(depth):
            g_desc(k, 0, k % NBUF).start()

        block_body(0, 0, first=True)

        def pair_body(m, carry):
            block_body(2 * m + 1, 1)
            block_body(2 * m + 2, 0)
            return carry

        lax.fori_loop(0, (nblk - 2) // 2, pair_body, 0)
        block_body(nblk - 1, (nblk - 1) % 2, last=True)
        s_desc(0, 0, (CPB - 1) % NBUF).wait()

        plsc.subcore_barrier()
        pltpu.sync_copy(acc.at[pl.ds(row0, rpt)],
                        qp_hbm.at[c, pl.ds(row0, rpt)])

    return agg_kernel


def _gscale_body(degp_ref, x_ref, w_ref, g_ref):
    deg = degp_ref[0, :] + degp_ref[1, :] + 1.0
    dinv = lax.rsqrt(deg)
    h = jnp.dot(x_ref[...], w_ref[...], preferred_element_type=jnp.float32)
    g_ref[...] = h * dinv[:, None]


def _combine_body(degp_ref, q_ref, g_ref, b_ref, o_ref):
    deg = degp_ref[0, :] + degp_ref[1, :] + 1.0
    dinv = lax.rsqrt(deg)
    ssum = q_ref[0] + q_ref[1] + g_ref[...]
    o_ref[...] = ssum * dinv[:, None] + b_ref[...]


def kernel(x, edge_index, W, b):
    n, nin = x.shape
    nout = W.shape[1]
    e = edge_index.shape[1]

    npm = max(BN, NS * 8)
    n_pad = -(-n // npm) * npm
    epw = -(-e // (NW * CPB * CH)) * (CPB * CH)
    ep = epw * NW
    nchunk = epw // CH
    nblk = nchunk // CPB

    src = edge_index[0].astype(jnp.int32)
    dst = edge_index[1].astype(jnp.int32)
    pad = ep - e
    if pad:
        extra = n + (jnp.arange(pad, dtype=jnp.int32) % (n_pad - n))
        src = jnp.concatenate([src, extra])
        dst = jnp.concatenate([dst, extra])
    dst_r = dst.reshape(NW, nchunk, CH)
    dst_r4 = dst.reshape(NW, nblk, CPB, CH)
    zrows = jnp.zeros((n_pad // NS, nout), jnp.float32)
    ones_ch = jnp.ones((CH,), jnp.float32)

    degp = _make_deg_kernel(n_pad, nchunk)(dst_r, ones_ch)

    g = pl.pallas_call(
        _gscale_body,
        grid=(n_pad // BN,),
        in_specs=[
            pl.BlockSpec((NC, BN), lambda i: (0, i)),
            pl.BlockSpec((BN, nin), lambda i: (i, 0)),
            pl.BlockSpec((nin, nout), lambda i: (0, 0)),
        ],
        out_specs=pl.BlockSpec((BN, nout), lambda i: (i, 0)),
        out_shape=jax.ShapeDtypeStruct((n_pad, nout), jnp.float32),
    )(degp, x, W)

    qp = _make_agg_kernel(n_pad, nout, epw, nchunk)(g, src, dst_r4, zrows)

    outp = pl.pallas_call(
        _combine_body,
        grid=(n_pad // BN,),
        in_specs=[
            pl.BlockSpec((NC, BN), lambda i: (0, i)),
            pl.BlockSpec((NC, BN, nout), lambda i: (0, i, 0)),
            pl.BlockSpec((BN, nout), lambda i: (i, 0)),
            pl.BlockSpec((1, nout), lambda i: (0, 0)),
        ],
        out_specs=pl.BlockSpec((BN, nout), lambda i: (i, 0)),
        out_shape=jax.ShapeDtypeStruct((n, nout), jnp.float32),
    )(degp, qp, g, b.reshape(1, nout))

    return outp


# speedup vs baseline: 1.1053x; 1.0415x over previous
import functools

import jax
import jax.numpy as jnp
from jax import lax
from jax.experimental import pallas as pl
from jax.experimental.pallas import tpu as pltpu
from jax.experimental.pallas import tpu_sc as plsc

NC = 2
NS = 16
NW = NC * NS
LANES = 16
CH = 80
NBUF = 3
BN = 1024


def _sc_mesh():
    return plsc.VectorSubcoreMesh(
        core_axis_name="c", subcore_axis_name="s",
        num_cores=NC, num_subcores=NS)


def _make_deg_kernel(n_pad, nchunk):
    rpt = n_pad // NS

    @functools.partial(
        pl.kernel,
        out_type=jax.ShapeDtypeStruct((NC, n_pad), jnp.float32),
        mesh=_sc_mesh(),
        scratch_types=[
            pltpu.VMEM((nchunk, CH), jnp.int32),
            pltpu.VMEM((CH,), jnp.float32),
            pltpu.VMEM((rpt,), jnp.float32),
            pltpu.VMEM_SHARED((n_pad,), jnp.float32),
            pltpu.SemaphoreType.DMA,
            pltpu.SemaphoreType.DMA,
        ],
    )
    def deg_kernel(dst_hbm, ones_hbm, degp_hbm, dstv, ones_v, zv, deg_sh,
                   dsem_a, dsem_b):
        c = lax.axis_index("c")
        s = lax.axis_index("s")
        wid = s * NC + c
        pltpu.sync_copy(dst_hbm.at[wid], dstv)
        pltpu.sync_copy(ones_hbm, ones_v)
        for i in range(rpt // LANES):
            zv[pl.ds(i * LANES, LANES)] = jnp.zeros((LANES,), jnp.float32)
        row0 = pl.multiple_of(s * rpt, 8)
        pltpu.sync_copy(zv, deg_sh.at[pl.ds(row0, rpt)])
        plsc.subcore_barrier()

        def d_desc(ci, sem):
            return pltpu.make_async_copy(ones_v, deg_sh.at[dstv.at[ci]], sem)

        def body(k, carry):
            c0 = k * 2
            d_desc(c0, dsem_a).start(add=True)
            d_desc(c0 + 1, dsem_b).start(add=True)
            d_desc(c0, dsem_a).wait()
            d_desc(c0 + 1, dsem_b).wait()
            return carry

        lax.fori_loop(0, nchunk // 2, body, 0)
        for ci in range(nchunk - nchunk % 2, nchunk):
            pltpu.sync_copy(ones_v, deg_sh.at[dstv.at[ci]], add=True)
        plsc.subcore_barrier()
        pltpu.sync_copy(deg_sh.at[pl.ds(row0, rpt)],
                        degp_hbm.at[c, pl.ds(row0, rpt)])

    return deg_kernel


CPB = 3


def _make_agg_kernel(n_pad, nout, epw, nchunk):
    rpt = n_pad // NS
    nblk = nchunk // CPB
    be = CPB * CH
    assert nblk >= 4 and nblk % 2 == 0 and CPB % NBUF == 0
    depth = NBUF - 1

    @functools.partial(
        pl.kernel,
        out_type=jax.ShapeDtypeStruct((NC, n_pad, nout), jnp.float32),
        mesh=_sc_mesh(),
        scratch_types=(
            [pltpu.VMEM((be,), jnp.int32)] * 2 +
            [pltpu.VMEM((CPB, CH), jnp.int32)] * 2 +
            [pltpu.VMEM((CH, nout), jnp.float32)] * NBUF +
            [pltpu.VMEM_SHARED((n_pad, nout), jnp.float32)] +
            [pltpu.SemaphoreType.DMA] * (2 * NBUF + 2)
        ),
    )
    def agg_kernel(g_hbm, src_hbm, dst_hbm, zrows_hbm, qp_hbm, *scr):
        srcv = list(scr[0:2])
        dstv = list(scr[2:4])
        rows = list(scr[4:4 + NBUF])
        acc = scr[4 + NBUF]
        gsem = list(scr[5 + NBUF:5 + 2 * NBUF])
        ssem = list(scr[5 + 2 * NBUF:5 + 3 * NBUF])
        isem = list(scr[5 + 3 * NBUF:7 + 3 * NBUF])
        c = lax.axis_index("c")
        s = lax.axis_index("s")
        wid = s * NC + c
        row0 = pl.multiple_of(s * rpt, 8)

        pltpu.sync_copy(zrows_hbm, acc.at[pl.ds(row0, rpt)])
        plsc.subcore_barrier()

        def g_desc(i, p, x):
            return pltpu.make_async_copy(
                g_hbm.at[srcv[p].at[pl.ds(i * CH, CH)]], rows[x], gsem[x])

        def s_desc(i, p, x):
            return pltpu.make_async_copy(
                rows[x], acc.at[dstv[p].at[i]], ssem[x])

        def stage_descs(sb, p):
            sbase = pl.multiple_of(wid * epw + sb * be, 8)
            return (pltpu.make_async_copy(
                        src_hbm.at[pl.ds(sbase, be)], srcv[p], isem[p]),
                    pltpu.make_async_copy(
                        dst_hbm.at[wid, sb], dstv[p], isem[p]))

        def block_body(sb, p, first=False, last=False):
            if not last:
                for d in stage_descs(sb + 1, 1 - p):
                    d.start()
            for i in range(CPB):
                x = i % NBUF
                g_desc(i, p, x).wait()
                s_desc(i, p, x).start(add=True)
                if not (first and i == 0):
                    s_desc(0, p, (i + NBUF - 1) % NBUF).wait()
                if i <= CPB - 1 - depth:
                    g_desc(i + depth, p, (i + depth) % NBUF).start()
                elif not last:
                    if i == CPB - depth:
                        for d in stage_descs(sb + 1, 1 - p):
                            d.wait()
                    g_desc(i + depth - CPB, 1 - p, (i + depth) % NBUF).start()

        for d in stage_descs(0, 0):
            d.start()
        for d in stage_descs(0, 0):
            d.wait()
        for k in range(depth):
            g_desc(k, 0, k % NBUF).start()

        block_body(0, 0, first=True)

        def pair_body(m, carry):
            block_body(2 * m + 1, 1)
            block_body(2 * m + 2, 0)
            return carry

        lax.fori_loop(0, (nblk - 2) // 2, pair_body, 0)
        block_body(nblk - 1, (nblk - 1) % 2, last=True)
        s_desc(0, 0, (CPB - 1) % NBUF).wait()

        plsc.subcore_barrier()
        pltpu.sync_copy(acc.at[pl.ds(row0, rpt)],
                        qp_hbm.at[c, pl.ds(row0, rpt)])

    return agg_kernel


def _gscale_body(degp_ref, x_ref, w_ref, g_ref):
    deg = degp_ref[0, :] + degp_ref[1, :] + 1.0
    dinv = lax.rsqrt(deg)
    h = jnp.dot(x_ref[...], w_ref[...], preferred_element_type=jnp.float32)
    g_ref[...] = h * dinv[:, None]


def _combine_body(degp_ref, q_ref, g_ref, b_ref, o_ref):
    deg = degp_ref[0, :] + degp_ref[1, :] + 1.0
    dinv = lax.rsqrt(deg)
    ssum = q_ref[0] + q_ref[1] + g_ref[...]
    o_ref[...] = ssum * dinv[:, None] + b_ref[...]


def kernel(x, edge_index, W, b):
    n, nin = x.shape
    nout = W.shape[1]
    e = edge_index.shape[1]

    npm = max(BN, NS * 8)
    n_pad = -(-n // npm) * npm
    epw = -(-e // (NW * CPB * CH)) * (CPB * CH)
    ep = epw * NW
    nchunk = epw // CH
    nblk = nchunk // CPB

    src = edge_index[0].astype(jnp.int32)
    dst = edge_index[1].astype(jnp.int32)
    pad = ep - e
    if pad:
        extra = n + (jnp.arange(pad, dtype=jnp.int32) % (n_pad - n))
        src = jnp.concatenate([src, extra])
        dst = jnp.concatenate([dst, extra])
    dst_r = dst.reshape(NW, nchunk, CH)
    dst_r4 = dst.reshape(NW, nblk, CPB, CH)
    zrows = jnp.zeros((n_pad // NS, nout), jnp.float32)
    ones_ch = jnp.ones((CH,), jnp.float32)

    degp = _make_deg_kernel(n_pad, nchunk)(dst_r, ones_ch)

    g = pl.pallas_call(
        _gscale_body,
        grid=(n_pad // BN,),
        in_specs=[
            pl.BlockSpec((NC, BN), lambda i: (0, i)),
            pl.BlockSpec((BN, nin), lambda i: (i, 0)),
            pl.BlockSpec((nin, nout), lambda i: (0, 0)),
        ],
        out_specs=pl.BlockSpec((BN, nout), lambda i: (i, 0)),
        out_shape=jax.ShapeDtypeStruct((n_pad, nout), jnp.float32),
    )(degp, x, W)

    qp = _make_agg_kernel(n_pad, nout, epw, nchunk)(g, src, dst_r4, zrows)

    outp = pl.pallas_call(
        _combine_body,
        grid=(n_pad // BN,),
        in_specs=[
            pl.BlockSpec((NC, BN), lambda i: (0, i)),
            pl.BlockSpec((NC, BN, nout), lambda i: (0, i, 0)),
            pl.BlockSpec((BN, nout), lambda i: (i, 0)),
            pl.BlockSpec((1, nout), lambda i: (0, 0)),
        ],
        out_specs=pl.BlockSpec((BN, nout), lambda i: (i, 0)),
        out_shape=jax.ShapeDtypeStruct((n, nout), jnp.float32),
    )(degp, qp, g, b.reshape(1, nout))

    return outp
